# baseline (speedup 1.0000x reference)
import functools

import jax
import jax.numpy as jnp
from jax import lax
from jax.experimental import pallas as pl
from jax.experimental.pallas import tpu as pltpu
from jax.experimental.pallas import tpu_sc as plsc

NUM_ENT = 1000000
EMBED_DIM = 64
BATCH = 4096
PAIRS = 50

_NC = 2
_NS = 16
_NW = _NC * _NS

_G = 128
_NGRP = BATCH * PAIRS * 2 // _G
_PER_W = _NGRP // _NW
_NBUF = 10
_NOUT = _PER_W // _NBUF

_S1_E = 1024
_S1_GRID = -(-NUM_ENT // _S1_E)


def _detile_body(x_ref, o_ref):
    x = x_ref[...]
    y = jnp.transpose(x)
    z = y.reshape(_S1_E // 2, 2, EMBED_DIM)
    o_ref[...] = jnp.concatenate([z[:, 0, :], z[:, 1, :]], axis=1)


def _detile(table_t):
    return pl.pallas_call(
        _detile_body,
        grid=(_S1_GRID,),
        in_specs=[pl.BlockSpec((EMBED_DIM, _S1_E), lambda i: (0, i))],
        out_specs=pl.BlockSpec((_S1_E // 2, 128), lambda i: (i, 0)),
        out_shape=jax.ShapeDtypeStruct((NUM_ENT // 2, 128), jnp.float32),
    )(table_t)


def _tile_xpose_body(x_ref, o_ref):
    z = x_ref[...]
    at = jnp.transpose(z[:, :EMBED_DIM])
    bt = jnp.transpose(z[:, EMBED_DIM:])
    y = jnp.concatenate([at, bt], axis=1)
    o_ref[...] = y.reshape(1, 1, 8, 1, 8, 128)


def _tile_xpose(gat2):
    return pl.pallas_call(
        _tile_xpose_body,
        grid=(_NGRP,),
        in_specs=[pl.BlockSpec((EMBED_DIM, 128), lambda i: (i, 0))],
        out_specs=pl.BlockSpec(
            (1, 1, 8, 1, 8, 128),
            lambda i: (i // 64, i % 2, 0, (i // 2) % 32, 0, 0),
        ),
        out_shape=jax.ShapeDtypeStruct((PAIRS, 2, 8, 32, 8, 128), jnp.float32),
    )(gat2)


def _sc_gather(table_lin, idx_g):
    mesh = plsc.VectorSubcoreMesh(core_axis_name="c", subcore_axis_name="s")

    @functools.partial(
        pl.kernel,
        mesh=mesh,
        out_type=jax.ShapeDtypeStruct((_NGRP, _G, EMBED_DIM), jnp.float32),
        scratch_types=(
            [pltpu.VMEM((_PER_W, _G), jnp.int32)]
            + [pltpu.VMEM((_G, EMBED_DIM), jnp.float32) for _ in range(_NBUF)]
            + [
                pltpu.SemaphoreType.DMA((_NBUF,)),
                pltpu.SemaphoreType.DMA((_NBUF,)),
            ]
        ),
        compiler_params=pltpu.CompilerParams(use_tc_tiling_on_sc=False),
    )
    def k(table_hbm, idx_hbm, out_hbm, idx_v, *rest):
        rows = rest[:_NBUF]
        gsem, osem = rest[_NBUF], rest[_NBUF + 1]

        wid = lax.axis_index("s") * _NC + lax.axis_index("c")
        gr0 = wid * _PER_W
        pltpu.sync_copy(idx_hbm.at[pl.ds(gr0, _PER_W)], idx_v)

        def gstart(b, i):
            pltpu.make_async_copy(
                table_hbm.at[idx_v.at[i]], rows[b], gsem.at[b]
            ).start()

        def gwait(b):
            pltpu.make_async_copy(
                table_hbm.at[idx_v.at[0]], rows[b], gsem.at[b]
            ).wait()

        def ostart(b, i):
            pltpu.make_async_copy(
                rows[b], out_hbm.at[gr0 + i], osem.at[b]
            ).start()

        def owait(b):
            pltpu.make_async_copy(
                rows[b], out_hbm.at[gr0], osem.at[b]
            ).wait()

        for b in range(_NBUF):
            gstart(b, b)

        def body(o, carry):
            for b in range(_NBUF):
                gwait(b)
                ostart(b, o * _NBUF + b)
            for b in range(_NBUF):
                owait(b)
                gstart(b, (o + 1) * _NBUF + b)
            return carry

        last = _NOUT - 1
        lax.fori_loop(0, last, body, 0)
        for b in range(_NBUF):
            gwait(b)
            ostart(b, last * _NBUF + b)
        for b in range(_NBUF):
            owait(b)

    return k(table_lin, idx_g)


def kernel(idx, embedding_weight):
    idx_g = (
        idx.reshape(32, 128, PAIRS, 2)
        .transpose(2, 0, 3, 1)
        .reshape(_NGRP, _G)
    )
    perm = jnp.arange(_G) // 2 + (jnp.arange(_G) % 2) * EMBED_DIM
    idx_g = idx_g[:, perm]
    table_t = jnp.transpose(embedding_weight)
    table_lin = _detile(table_t).reshape(NUM_ENT, EMBED_DIM)
    gat3 = _sc_gather(table_lin, idx_g)
    out6 = _tile_xpose(gat3.reshape(_NGRP * EMBED_DIM, _G))
    return (
        out6.transpose(3, 5, 0, 1, 2, 4)
        .reshape(BATCH, PAIRS, 2, EMBED_DIM)
    )


# speedup vs baseline: 4.1484x; 4.1484x over previous
import functools

import jax
import jax.numpy as jnp
from jax import lax
from jax.experimental import pallas as pl
from jax.experimental.pallas import tpu as pltpu
from jax.experimental.pallas import tpu_sc as plsc

NUM_ENT = 1000000
EMBED_DIM = 64
BATCH = 4096
PAIRS = 50

_NC = 2
_NS = 16
_NW = _NC * _NS

_G = 128
_NGRP = BATCH * PAIRS * 2 // _G
_PER_W = _NGRP // _NW
_NBUF = 10
_NOUT = _PER_W // _NBUF

_S1_E = 8192
_S1_GRID = -(-NUM_ENT // _S1_E)


def _detile_body(x_ref, o_ref):
    x = x_ref[...]
    y = jnp.transpose(x)
    z = y.reshape(_S1_E // 2, 2, EMBED_DIM)
    o_ref[...] = jnp.concatenate([z[:, 0, :], z[:, 1, :]], axis=1)


def _detile(table_t):
    return pl.pallas_call(
        _detile_body,
        grid=(_S1_GRID,),
        in_specs=[pl.BlockSpec((EMBED_DIM, _S1_E), lambda i: (0, i))],
        out_specs=pl.BlockSpec((_S1_E // 2, 128), lambda i: (i, 0)),
        out_shape=jax.ShapeDtypeStruct((NUM_ENT // 2, 128), jnp.float32),
    )(table_t)


def _tile_xpose_body(x_ref, o_ref):
    x = x_ref[0, :, 0]
    at = jnp.transpose(x[:, :, :EMBED_DIM], (0, 2, 1))
    bt = jnp.transpose(x[:, :, EMBED_DIM:], (0, 2, 1))
    y = jnp.concatenate([at, bt], axis=2)
    y4 = y.reshape(32, 8, 8, 128)
    for g in range(8):
        o_ref[0, 0, g, :, :, :] = y4[:, g, :, :]


def _tile_xpose(gat5):
    return pl.pallas_call(
        _tile_xpose_body,
        grid=(PAIRS, 2),
        in_specs=[
            pl.BlockSpec(
                (1, 32, 1, EMBED_DIM, 128), lambda p, l: (p, 0, l, 0, 0)
            )
        ],
        out_specs=pl.BlockSpec(
            (1, 1, 8, 32, 8, 128), lambda p, l: (p, l, 0, 0, 0, 0)
        ),
        out_shape=jax.ShapeDtypeStruct((PAIRS, 2, 8, 32, 8, 128), jnp.float32),
    )(gat5)


def _sc_gather(table_lin, idx_g):
    mesh = plsc.VectorSubcoreMesh(core_axis_name="c", subcore_axis_name="s")

    @functools.partial(
        pl.kernel,
        mesh=mesh,
        out_type=jax.ShapeDtypeStruct((_NGRP, _G, EMBED_DIM), jnp.float32),
        scratch_types=(
            [pltpu.VMEM((_PER_W, _G), jnp.int32)]
            + [pltpu.VMEM((_G, EMBED_DIM), jnp.float32) for _ in range(_NBUF)]
            + [
                pltpu.SemaphoreType.DMA((_NBUF,)),
                pltpu.SemaphoreType.DMA((_NBUF,)),
            ]
        ),
        compiler_params=pltpu.CompilerParams(use_tc_tiling_on_sc=False),
    )
    def k(table_hbm, idx_hbm, out_hbm, idx_v, *rest):
        rows = rest[:_NBUF]
        gsem, osem = rest[_NBUF], rest[_NBUF + 1]

        wid = lax.axis_index("s") * _NC + lax.axis_index("c")
        gr0 = wid * _PER_W
        pltpu.sync_copy(idx_hbm.at[pl.ds(gr0, _PER_W)], idx_v)

        def gstart(b, i):
            pltpu.make_async_copy(
                table_hbm.at[idx_v.at[i]], rows[b], gsem.at[b]
            ).start()

        def gwait(b):
            pltpu.make_async_copy(
                table_hbm.at[idx_v.at[0]], rows[b], gsem.at[b]
            ).wait()

        def ostart(b, i):
            pltpu.make_async_copy(
                rows[b], out_hbm.at[gr0 + i], osem.at[b]
            ).start()

        def owait(b):
            pltpu.make_async_copy(
                rows[b], out_hbm.at[gr0], osem.at[b]
            ).wait()

        for b in range(_NBUF):
            gstart(b, b)

        def body(o, carry):
            for b in range(_NBUF):
                gwait(b)
                ostart(b, o * _NBUF + b)
            for b in range(_NBUF):
                owait(b)
                gstart(b, (o + 1) * _NBUF + b)
            return carry

        last = _NOUT - 1
        lax.fori_loop(0, last, body, 0)
        for b in range(_NBUF):
            gwait(b)
            ostart(b, last * _NBUF + b)
        for b in range(_NBUF):
            owait(b)

    return k(table_lin, idx_g)


def kernel(idx, embedding_weight):
    idx_g = (
        idx.reshape(32, 128, PAIRS, 2)
        .transpose(2, 0, 3, 1)
        .reshape(_NGRP, _G)
    )
    perm = jnp.arange(_G) // 2 + (jnp.arange(_G) % 2) * EMBED_DIM
    idx_g = idx_g[:, perm]
    table_t = jnp.transpose(embedding_weight)
    table_lin = _detile(table_t).reshape(NUM_ENT, EMBED_DIM)
    gat3 = _sc_gather(table_lin, idx_g)
    out6 = _tile_xpose(gat3.reshape(PAIRS, 32, 2, EMBED_DIM, 128))
    return (
        out6.transpose(3, 5, 0, 1, 2, 4)
        .reshape(BATCH, PAIRS, 2, EMBED_DIM)
    )


# speedup vs baseline: 5.3646x; 1.2932x over previous
import functools

import jax
import jax.numpy as jnp
from jax import lax
from jax.experimental import pallas as pl
from jax.experimental.pallas import tpu as pltpu
from jax.experimental.pallas import tpu_sc as plsc

NUM_ENT = 1000000
EMBED_DIM = 64
BATCH = 4096
PAIRS = 50

_NC = 2
_NS = 16
_NW = _NC * _NS

_G = 128
_NGRP = BATCH * PAIRS * 2 // _G
_PER_W = _NGRP // _NW
_NBUF = 10
_NOUT = _PER_W // _NBUF

_H = 500224
_S1_E = 4096
_S1_GRID = -(-_H // _S1_E)


def _detile_body(x1_ref, x2_ref, o_ref):
    xc = jnp.concatenate([x1_ref[...], x2_ref[...]], axis=0)
    o_ref[...] = jnp.transpose(xc)


def _detile(table_t, table_t2):
    return pl.pallas_call(
        _detile_body,
        grid=(_S1_GRID,),
        in_specs=[
            pl.BlockSpec((EMBED_DIM, _S1_E), lambda i: (0, i)),
            pl.BlockSpec((EMBED_DIM, _S1_E), lambda i: (0, i)),
        ],
        out_specs=pl.BlockSpec((_S1_E, 128), lambda i: (i, 0)),
        out_shape=jax.ShapeDtypeStruct((_H, 128), jnp.float32),
    )(table_t, table_t2)


def _tile_xpose_body(x_ref, o_ref):
    x = x_ref[0, :, 0]
    w = jnp.concatenate([x[:, :, :EMBED_DIM], x[:, :, EMBED_DIM:]], axis=1)
    y = jnp.transpose(w, (0, 2, 1))
    y4 = y.reshape(32, 8, 8, 128)
    for g in range(8):
        o_ref[0, 0, g, :, :, :] = y4[:, g, :, :]


def _tile_xpose(gat5):
    return pl.pallas_call(
        _tile_xpose_body,
        grid=(PAIRS, 2),
        in_specs=[
            pl.BlockSpec(
                (1, 32, 1, EMBED_DIM, 128), lambda p, l: (p, 0, l, 0, 0)
            )
        ],
        out_specs=pl.BlockSpec(
            (1, 1, 8, 32, 8, 128), lambda p, l: (p, l, 0, 0, 0, 0)
        ),
        out_shape=jax.ShapeDtypeStruct((PAIRS, 2, 8, 32, 8, 128), jnp.float32),
    )(gat5)


def _sc_gather(table_lin, idx_g):
    mesh = plsc.VectorSubcoreMesh(core_axis_name="c", subcore_axis_name="s")

    @functools.partial(
        pl.kernel,
        mesh=mesh,
        out_type=jax.ShapeDtypeStruct((_NGRP, _G, EMBED_DIM), jnp.float32),
        scratch_types=(
            [pltpu.VMEM((_PER_W, _G), jnp.int32)]
            + [pltpu.VMEM((_G, EMBED_DIM), jnp.float32) for _ in range(_NBUF)]
            + [
                pltpu.SemaphoreType.DMA((_NBUF,)),
                pltpu.SemaphoreType.DMA((_NBUF,)),
            ]
        ),
        compiler_params=pltpu.CompilerParams(use_tc_tiling_on_sc=False),
    )
    def k(table_hbm, idx_hbm, out_hbm, idx_v, *rest):
        rows = rest[:_NBUF]
        gsem, osem = rest[_NBUF], rest[_NBUF + 1]

        wid = lax.axis_index("s") * _NC + lax.axis_index("c")
        gr0 = wid * _PER_W
        pltpu.sync_copy(idx_hbm.at[pl.ds(gr0, _PER_W)], idx_v)

        def gstart(b, i):
            pltpu.make_async_copy(
                table_hbm.at[idx_v.at[i]], rows[b], gsem.at[b]
            ).start()

        def gwait(b):
            pltpu.make_async_copy(
                table_hbm.at[idx_v.at[0]], rows[b], gsem.at[b]
            ).wait()

        def ostart(b, i):
            pltpu.make_async_copy(
                rows[b], out_hbm.at[gr0 + i], osem.at[b]
            ).start()

        def owait(b):
            pltpu.make_async_copy(
                rows[b], out_hbm.at[gr0], osem.at[b]
            ).wait()

        for b in range(_NBUF):
            gstart(b, b)

        def body(o, carry):
            for b in range(_NBUF):
                gwait(b)
                ostart(b, o * _NBUF + b)
            for b in range(_NBUF):
                owait(b)
                gstart(b, (o + 1) * _NBUF + b)
            return carry

        last = _NOUT - 1
        lax.fori_loop(0, last, body, 0)
        for b in range(_NBUF):
            gwait(b)
            ostart(b, last * _NBUF + b)
        for b in range(_NBUF):
            owait(b)

    return k(table_lin, idx_g)


def kernel(idx, embedding_weight):
    idx_g = (
        idx.reshape(32, 128, PAIRS, 2)
        .transpose(2, 0, 3, 1)
        .reshape(_NGRP, _G)
    )
    perm = jnp.arange(_G) // 2 + (jnp.arange(_G) % 2) * EMBED_DIM
    idx_g = idx_g[:, perm]
    idx_g = 2 * idx_g - jnp.where(idx_g >= _H, 2 * _H - 1, 0)

    table_t = jnp.transpose(embedding_weight)
    table_t2 = table_t[:, _H:]
    table_lin = _detile(table_t, table_t2).reshape(2 * _H, EMBED_DIM)
    gat3 = _sc_gather(table_lin, idx_g)
    out6 = _tile_xpose(gat3.reshape(PAIRS, 32, 2, EMBED_DIM, 128))
    return (
        out6.transpose(3, 5, 0, 1, 2, 4)
        .reshape(BATCH, PAIRS, 2, EMBED_DIM)
    )


# speedup vs baseline: 5.6598x; 1.0550x over previous
import functools

import jax
import jax.numpy as jnp
from jax import lax
from jax.experimental import pallas as pl
from jax.experimental.pallas import tpu as pltpu
from jax.experimental.pallas import tpu_sc as plsc

NUM_ENT = 1000000
EMBED_DIM = 64
BATCH = 4096
PAIRS = 50

_NC = 2
_NS = 16
_NW = _NC * _NS

_G = 128
_NGRP = BATCH * PAIRS * 2 // _G
_PER_W = _NGRP // _NW
_NBUF = 10
_NOUT = _PER_W // _NBUF

_H = 500224
_S1_E = 8192
_S1_GRID = -(-_H // _S1_E)


def _detile_body(x1_ref, x2_ref, o_ref):
    xc = jnp.concatenate([x1_ref[...], x2_ref[...]], axis=0)
    o_ref[...] = jnp.transpose(xc)


def _detile(table_t, table_t2):
    return pl.pallas_call(
        _detile_body,
        grid=(_S1_GRID,),
        in_specs=[
            pl.BlockSpec((EMBED_DIM, _S1_E), lambda i: (0, i)),
            pl.BlockSpec((EMBED_DIM, _S1_E), lambda i: (0, i)),
        ],
        out_specs=pl.BlockSpec((_S1_E, 128), lambda i: (i, 0)),
        out_shape=jax.ShapeDtypeStruct((_H, 128), jnp.float32),
    )(table_t, table_t2)


def _tile_xpose_body(x_ref, o_ref):
    x = x_ref[0, :, 0]
    w = jnp.concatenate([x[:, :, :EMBED_DIM], x[:, :, EMBED_DIM:]], axis=1)
    y = jnp.transpose(w, (0, 2, 1))
    y4 = y.reshape(32, 8, 8, 128)
    for g in range(8):
        o_ref[0, 0, g, :, :, :] = y4[:, g, :, :]


def _tile_xpose(gat5):
    return pl.pallas_call(
        _tile_xpose_body,
        grid=(PAIRS, 2),
        in_specs=[
            pl.BlockSpec(
                (1, 32, 1, EMBED_DIM, 128), lambda p, l: (p, 0, l, 0, 0)
            )
        ],
        out_specs=pl.BlockSpec(
            (1, 1, 8, 32, 8, 128), lambda p, l: (p, l, 0, 0, 0, 0)
        ),
        out_shape=jax.ShapeDtypeStruct((PAIRS, 2, 8, 32, 8, 128), jnp.float32),
    )(gat5)


def _sc_gather(table_lin, idx_g):
    mesh = plsc.VectorSubcoreMesh(core_axis_name="c", subcore_axis_name="s")

    @functools.partial(
        pl.kernel,
        mesh=mesh,
        out_type=jax.ShapeDtypeStruct((_NGRP, _G, EMBED_DIM), jnp.float32),
        scratch_types=(
            [pltpu.VMEM((_PER_W, _G), jnp.int32)]
            + [pltpu.VMEM((_G, EMBED_DIM), jnp.float32) for _ in range(_NBUF)]
            + [
                pltpu.SemaphoreType.DMA((_NBUF,)),
                pltpu.SemaphoreType.DMA((_NBUF,)),
            ]
        ),
        compiler_params=pltpu.CompilerParams(use_tc_tiling_on_sc=False),
    )
    def k(table_hbm, idx_hbm, out_hbm, idx_v, *rest):
        rows = rest[:_NBUF]
        gsem, osem = rest[_NBUF], rest[_NBUF + 1]

        wid = lax.axis_index("s") * _NC + lax.axis_index("c")
        gr0 = wid * _PER_W
        pltpu.sync_copy(idx_hbm.at[pl.ds(gr0, _PER_W)], idx_v)

        def gstart(b, i):
            pltpu.make_async_copy(
                table_hbm.at[idx_v.at[i]], rows[b], gsem.at[b]
            ).start()

        def gwait(b):
            pltpu.make_async_copy(
                table_hbm.at[idx_v.at[0]], rows[b], gsem.at[b]
            ).wait()

        def ostart(b, i):
            pltpu.make_async_copy(
                rows[b], out_hbm.at[gr0 + i], osem.at[b]
            ).start()

        def owait(b):
            pltpu.make_async_copy(
                rows[b], out_hbm.at[gr0], osem.at[b]
            ).wait()

        for b in range(_NBUF):
            gstart(b, b)

        def body(o, carry):
            for b in range(_NBUF):
                gwait(b)
                ostart(b, o * _NBUF + b)
            for b in range(_NBUF):
                owait(b)
                gstart(b, (o + 1) * _NBUF + b)
            return carry

        last = _NOUT - 1
        lax.fori_loop(0, last, body, 0)
        for b in range(_NBUF):
            gwait(b)
            ostart(b, last * _NBUF + b)
        for b in range(_NBUF):
            owait(b)

    return k(table_lin, idx_g)


def kernel(idx, embedding_weight):
    idx_g = (
        idx.reshape(32, 128, PAIRS, 2)
        .transpose(2, 0, 3, 1)
        .reshape(_NGRP, _G)
    )
    perm = jnp.arange(_G) // 2 + (jnp.arange(_G) % 2) * EMBED_DIM
    idx_g = idx_g[:, perm]
    idx_g = 2 * idx_g - jnp.where(idx_g >= _H, 2 * _H - 1, 0)

    table_t = jnp.transpose(embedding_weight)
    table_t2 = table_t[:, _H:]
    table_lin = _detile(table_t, table_t2).reshape(2 * _H, EMBED_DIM)
    gat3 = _sc_gather(table_lin, idx_g)
    out6 = _tile_xpose(gat3.reshape(PAIRS, 32, 2, EMBED_DIM, 128))
    return (
        out6.transpose(3, 5, 0, 1, 2, 4)
        .reshape(BATCH, PAIRS, 2, EMBED_DIM)
    )


# speedup vs baseline: 6.2745x; 1.1086x over previous
import functools

import jax
import jax.numpy as jnp
from jax import lax
from jax.experimental import pallas as pl
from jax.experimental.pallas import tpu as pltpu
from jax.experimental.pallas import tpu_sc as plsc

NUM_ENT = 1000000
EMBED_DIM = 64
BATCH = 4096
PAIRS = 50

_NC = 2
_NS = 16
_NW = _NC * _NS

_G = 128
_NGRP = BATCH * PAIRS * 2 // _G
_PER_W = _NGRP // _NW
_NBUF = 10
_NOUT = _PER_W // _NBUF

_H = 500224
_S1_E = 16384
_S1_GRID = -(-_H // _S1_E)


def _detile_body(x1_ref, x2_ref, o_ref):
    xc = jnp.concatenate([x1_ref[...], x2_ref[...]], axis=0)
    o_ref[...] = jnp.transpose(xc)


def _detile(table_t, table_t2):
    return pl.pallas_call(
        _detile_body,
        grid=(_S1_GRID,),
        in_specs=[
            pl.BlockSpec((EMBED_DIM, _S1_E), lambda i: (0, i)),
            pl.BlockSpec((EMBED_DIM, _S1_E), lambda i: (0, i)),
        ],
        out_specs=pl.BlockSpec((_S1_E, 128), lambda i: (i, 0)),
        out_shape=jax.ShapeDtypeStruct((_H, 128), jnp.float32),
    )(table_t, table_t2)


def _tile_xpose_body(x_ref, o_ref):
    x = x_ref[:, :, 0].reshape(_S3_PB * 32, EMBED_DIM, 128)
    w = jnp.concatenate([x[:, :, :EMBED_DIM], x[:, :, EMBED_DIM:]], axis=1)
    y = jnp.transpose(w, (0, 2, 1))
    y4 = y.reshape(_S3_PB, 32, 8, 8, 128)
    for g in range(8):
        o_ref[:, 0, g, :, :, :] = y4[:, :, g, :, :]


_S3_PB = 5


def _tile_xpose(gat5):
    return pl.pallas_call(
        _tile_xpose_body,
        grid=(PAIRS // _S3_PB, 2),
        in_specs=[
            pl.BlockSpec(
                (_S3_PB, 32, 1, EMBED_DIM, 128), lambda p, l: (p, 0, l, 0, 0)
            )
        ],
        out_specs=pl.BlockSpec(
            (_S3_PB, 1, 8, 32, 8, 128), lambda p, l: (p, l, 0, 0, 0, 0)
        ),
        out_shape=jax.ShapeDtypeStruct((PAIRS, 2, 8, 32, 8, 128), jnp.float32),
    )(gat5)


def _sc_gather(table_lin, idx_g):
    mesh = plsc.VectorSubcoreMesh(core_axis_name="c", subcore_axis_name="s")

    @functools.partial(
        pl.kernel,
        mesh=mesh,
        out_type=jax.ShapeDtypeStruct((_NGRP, _G, EMBED_DIM), jnp.float32),
        scratch_types=(
            [pltpu.VMEM((_PER_W, _G), jnp.int32)]
            + [pltpu.VMEM((_G, EMBED_DIM), jnp.float32) for _ in range(_NBUF)]
            + [
                pltpu.SemaphoreType.DMA((_NBUF,)),
                pltpu.SemaphoreType.DMA((_NBUF,)),
            ]
        ),
        compiler_params=pltpu.CompilerParams(use_tc_tiling_on_sc=False),
    )
    def k(table_hbm, idx_hbm, out_hbm, idx_v, *rest):
        rows = rest[:_NBUF]
        gsem, osem = rest[_NBUF], rest[_NBUF + 1]

        wid = lax.axis_index("s") * _NC + lax.axis_index("c")
        gr0 = wid * _PER_W
        pltpu.sync_copy(idx_hbm.at[pl.ds(gr0, _PER_W)], idx_v)

        def gstart(b, i):
            pltpu.make_async_copy(
                table_hbm.at[idx_v.at[i]], rows[b], gsem.at[b]
            ).start()

        def gwait(b):
            pltpu.make_async_copy(
                table_hbm.at[idx_v.at[0]], rows[b], gsem.at[b]
            ).wait()

        def ostart(b, i):
            pltpu.make_async_copy(
                rows[b], out_hbm.at[gr0 + i], osem.at[b]
            ).start()

        def owait(b):
            pltpu.make_async_copy(
                rows[b], out_hbm.at[gr0], osem.at[b]
            ).wait()

        for b in range(_NBUF):
            gstart(b, b)

        def body(o, carry):
            for b in range(_NBUF):
                gwait(b)
                ostart(b, o * _NBUF + b)
            for b in range(_NBUF):
                owait(b)
                gstart(b, (o + 1) * _NBUF + b)
            return carry

        last = _NOUT - 1
        lax.fori_loop(0, last, body, 0)
        for b in range(_NBUF):
            gwait(b)
            ostart(b, last * _NBUF + b)
        for b in range(_NBUF):
            owait(b)

    return k(table_lin, idx_g)


def kernel(idx, embedding_weight):
    idx_g = (
        idx.reshape(32, 128, PAIRS, 2)
        .transpose(2, 0, 3, 1)
        .reshape(_NGRP, _G)
    )
    perm = jnp.arange(_G) // 2 + (jnp.arange(_G) % 2) * EMBED_DIM
    idx_g = idx_g[:, perm]
    idx_g = 2 * idx_g - jnp.where(idx_g >= _H, 2 * _H - 1, 0)

    table_t = jnp.transpose(embedding_weight)
    table_t2 = table_t[:, _H:]
    table_lin = _detile(table_t, table_t2).reshape(2 * _H, EMBED_DIM)
    gat3 = _sc_gather(table_lin, idx_g)
    out6 = _tile_xpose(gat3.reshape(PAIRS, 32, 2, EMBED_DIM, 128))
    return (
        out6.transpose(3, 5, 0, 1, 2, 4)
        .reshape(BATCH, PAIRS, 2, EMBED_DIM)
    )


# speedup vs baseline: 6.2897x; 1.0024x over previous
import functools

import jax
import jax.numpy as jnp
from jax import lax
from jax.experimental import pallas as pl
from jax.experimental.pallas import tpu as pltpu
from jax.experimental.pallas import tpu_sc as plsc

NUM_ENT = 1000000
EMBED_DIM = 64
BATCH = 4096
PAIRS = 50

_NC = 2
_NS = 16
_NW = _NC * _NS

_G = 128
_NGRP = BATCH * PAIRS * 2 // _G
_PER_W = _NGRP // _NW
_NBUF = 10
_NOUT = _PER_W // _NBUF

_H = 500224
_S1_E = 24576
_S1_GRID = -(-_H // _S1_E)


def _detile_body(x1_ref, x2_ref, o_ref):
    xc = jnp.concatenate([x1_ref[...], x2_ref[...]], axis=0)
    o_ref[...] = jnp.transpose(xc)


def _detile(table_t, table_t2):
    return pl.pallas_call(
        _detile_body,
        grid=(_S1_GRID,),
        in_specs=[
            pl.BlockSpec((EMBED_DIM, _S1_E), lambda i: (0, i)),
            pl.BlockSpec((EMBED_DIM, _S1_E), lambda i: (0, i)),
        ],
        out_specs=pl.BlockSpec((_S1_E, 128), lambda i: (i, 0)),
        out_shape=jax.ShapeDtypeStruct((_H, 128), jnp.float32),
    )(table_t, table_t2)


def _tile_xpose_body(x_ref, o_ref):
    x = x_ref[:, :, 0].reshape(_S3_PB * 32, EMBED_DIM, 128)
    w = jnp.concatenate([x[:, :, :EMBED_DIM], x[:, :, EMBED_DIM:]], axis=1)
    y = jnp.transpose(w, (0, 2, 1))
    y4 = y.reshape(_S3_PB, 32, 8, 8, 128)
    for g in range(8):
        o_ref[:, 0, g, :, :, :] = y4[:, :, g, :, :]


_S3_PB = 10


def _tile_xpose(gat5):
    return pl.pallas_call(
        _tile_xpose_body,
        grid=(PAIRS // _S3_PB, 2),
        in_specs=[
            pl.BlockSpec(
                (_S3_PB, 32, 1, EMBED_DIM, 128), lambda p, l: (p, 0, l, 0, 0)
            )
        ],
        out_specs=pl.BlockSpec(
            (_S3_PB, 1, 8, 32, 8, 128), lambda p, l: (p, l, 0, 0, 0, 0)
        ),
        out_shape=jax.ShapeDtypeStruct((PAIRS, 2, 8, 32, 8, 128), jnp.float32),
    )(gat5)


def _sc_gather(table_lin, idx_g):
    mesh = plsc.VectorSubcoreMesh(core_axis_name="c", subcore_axis_name="s")

    @functools.partial(
        pl.kernel,
        mesh=mesh,
        out_type=jax.ShapeDtypeStruct((_NGRP, _G, EMBED_DIM), jnp.float32),
        scratch_types=(
            [pltpu.VMEM((_PER_W, _G), jnp.int32)]
            + [pltpu.VMEM((_G, EMBED_DIM), jnp.float32) for _ in range(_NBUF)]
            + [
                pltpu.SemaphoreType.DMA((_NBUF,)),
                pltpu.SemaphoreType.DMA((_NBUF,)),
            ]
        ),
        compiler_params=pltpu.CompilerParams(use_tc_tiling_on_sc=False),
    )
    def k(table_hbm, idx_hbm, out_hbm, idx_v, *rest):
        rows = rest[:_NBUF]
        gsem, osem = rest[_NBUF], rest[_NBUF + 1]

        wid = lax.axis_index("s") * _NC + lax.axis_index("c")
        gr0 = wid * _PER_W
        pltpu.sync_copy(idx_hbm.at[pl.ds(gr0, _PER_W)], idx_v)

        def gstart(b, i):
            pltpu.make_async_copy(
                table_hbm.at[idx_v.at[i]], rows[b], gsem.at[b]
            ).start()

        def gwait(b):
            pltpu.make_async_copy(
                table_hbm.at[idx_v.at[0]], rows[b], gsem.at[b]
            ).wait()

        def ostart(b, i):
            pltpu.make_async_copy(
                rows[b], out_hbm.at[gr0 + i], osem.at[b]
            ).start()

        def owait(b):
            pltpu.make_async_copy(
                rows[b], out_hbm.at[gr0], osem.at[b]
            ).wait()

        for b in range(_NBUF):
            gstart(b, b)

        def body(o, carry):
            for b in range(_NBUF):
                gwait(b)
                ostart(b, o * _NBUF + b)
            for b in range(_NBUF):
                owait(b)
                gstart(b, (o + 1) * _NBUF + b)
            return carry

        last = _NOUT - 1
        lax.fori_loop(0, last, body, 0)
        for b in range(_NBUF):
            gwait(b)
            ostart(b, last * _NBUF + b)
        for b in range(_NBUF):
            owait(b)

    return k(table_lin, idx_g)


def kernel(idx, embedding_weight):
    idx_g = (
        idx.reshape(32, 128, PAIRS, 2)
        .transpose(2, 0, 3, 1)
        .reshape(_NGRP, _G)
    )
    perm = jnp.arange(_G) // 2 + (jnp.arange(_G) % 2) * EMBED_DIM
    idx_g = idx_g[:, perm]
    idx_g = 2 * idx_g - jnp.where(idx_g >= _H, 2 * _H - 1, 0)

    table_t = jnp.transpose(embedding_weight)
    table_t2 = table_t[:, _H:]
    table_lin = _detile(table_t, table_t2).reshape(2 * _H, EMBED_DIM)
    gat3 = _sc_gather(table_lin, idx_g)
    out6 = _tile_xpose(gat3.reshape(PAIRS, 32, 2, EMBED_DIM, 128))
    return (
        out6.transpose(3, 5, 0, 1, 2, 4)
        .reshape(BATCH, PAIRS, 2, EMBED_DIM)
    )


# speedup vs baseline: 6.3910x; 1.0161x over previous
import functools

import jax
import jax.numpy as jnp
from jax import lax
from jax.experimental import pallas as pl
from jax.experimental.pallas import tpu as pltpu
from jax.experimental.pallas import tpu_sc as plsc

NUM_ENT = 1000000
EMBED_DIM = 64
BATCH = 4096
PAIRS = 50

_NC = 2
_NS = 16
_NW = _NC * _NS

_G = 128
_NGRP = BATCH * PAIRS * 2 // _G
_PER_W = _NGRP // _NW
_NBUF = 10
_NOUT = _PER_W // _NBUF

_H = 500224
_S1_E = 24576
_S1_GRID = -(-_H // _S1_E)


def _detile_body(x1_ref, x2_ref, o_ref):
    xc = jnp.concatenate([x1_ref[...], x2_ref[...]], axis=0)
    o_ref[...] = jnp.transpose(xc)


def _detile(table_t, table_t2):
    return pl.pallas_call(
        _detile_body,
        grid=(_S1_GRID,),
        in_specs=[
            pl.BlockSpec((EMBED_DIM, _S1_E), lambda i: (0, i)),
            pl.BlockSpec((EMBED_DIM, _S1_E), lambda i: (0, i)),
        ],
        out_specs=pl.BlockSpec((_S1_E, 128), lambda i: (i, 0)),
        out_shape=jax.ShapeDtypeStruct((_H, 128), jnp.float32),
    )(table_t, table_t2)


def _tile_xpose_body(x_ref, o_ref):
    x = x_ref[:, :, 0].reshape(_S3_PB * 32, EMBED_DIM, 128)
    w = jnp.concatenate([x[:, :, :EMBED_DIM], x[:, :, EMBED_DIM:]], axis=1)
    y = jnp.transpose(w, (0, 2, 1))
    y4 = y.reshape(_S3_PB, 32, 8, 8, 128)
    for g in range(8):
        o_ref[:, 0, g, :, :, :] = y4[:, :, g, :, :]


_S3_PB = 5
_HP = PAIRS // 2


def _tile_xpose_a(gat5):
    return pl.pallas_call(
        _tile_xpose_body,
        grid=(_HP // _S3_PB, 2),
        in_specs=[
            pl.BlockSpec(
                (_S3_PB, 32, 1, EMBED_DIM, 128), lambda p, l: (p, 0, l, 0, 0)
            )
        ],
        out_specs=pl.BlockSpec(
            (_S3_PB, 1, 8, 32, 8, 128), lambda p, l: (p, l, 0, 0, 0, 0)
        ),
        out_shape=jax.ShapeDtypeStruct((PAIRS, 2, 8, 32, 8, 128), jnp.float32),
    )(gat5)


def _tile_xpose_b(gat5, prev):
    def body(x_ref, prev_ref, o_ref):
        _tile_xpose_body(x_ref, o_ref)

    off = _HP // _S3_PB
    return pl.pallas_call(
        body,
        grid=(_HP // _S3_PB, 2),
        in_specs=[
            pl.BlockSpec(
                (_S3_PB, 32, 1, EMBED_DIM, 128), lambda p, l: (p, 0, l, 0, 0)
            ),
            pl.BlockSpec(memory_space=pl.ANY),
        ],
        out_specs=pl.BlockSpec(
            (_S3_PB, 1, 8, 32, 8, 128), lambda p, l: (p + off, l, 0, 0, 0, 0)
        ),
        out_shape=jax.ShapeDtypeStruct((PAIRS, 2, 8, 32, 8, 128), jnp.float32),
        input_output_aliases={1: 0},
    )(gat5, prev)


def _sc_gather(table_lin, idx_g):
    ngrp = idx_g.shape[0]
    per_w = ngrp // _NW
    nout = per_w // _NBUF
    mesh = plsc.VectorSubcoreMesh(core_axis_name="c", subcore_axis_name="s")

    @functools.partial(
        pl.kernel,
        mesh=mesh,
        out_type=jax.ShapeDtypeStruct((ngrp, _G, EMBED_DIM), jnp.float32),
        scratch_types=(
            [pltpu.VMEM((per_w, _G), jnp.int32)]
            + [pltpu.VMEM((_G, EMBED_DIM), jnp.float32) for _ in range(_NBUF)]
            + [
                pltpu.SemaphoreType.DMA((_NBUF,)),
                pltpu.SemaphoreType.DMA((_NBUF,)),
            ]
        ),
        compiler_params=pltpu.CompilerParams(use_tc_tiling_on_sc=False),
    )
    def k(table_hbm, idx_hbm, out_hbm, idx_v, *rest):
        rows = rest[:_NBUF]
        gsem, osem = rest[_NBUF], rest[_NBUF + 1]

        wid = lax.axis_index("s") * _NC + lax.axis_index("c")
        gr0 = wid * per_w
        pltpu.sync_copy(idx_hbm.at[pl.ds(gr0, per_w)], idx_v)

        def gstart(b, i):
            pltpu.make_async_copy(
                table_hbm.at[idx_v.at[i]], rows[b], gsem.at[b]
            ).start()

        def gwait(b):
            pltpu.make_async_copy(
                table_hbm.at[idx_v.at[0]], rows[b], gsem.at[b]
            ).wait()

        def ostart(b, i):
            pltpu.make_async_copy(
                rows[b], out_hbm.at[gr0 + i], osem.at[b]
            ).start()

        def owait(b):
            pltpu.make_async_copy(
                rows[b], out_hbm.at[gr0], osem.at[b]
            ).wait()

        for b in range(_NBUF):
            gstart(b, b)

        def body(o, carry):
            for b in range(_NBUF):
                gwait(b)
                ostart(b, o * _NBUF + b)
            for b in range(_NBUF):
                owait(b)
                gstart(b, (o + 1) * _NBUF + b)
            return carry

        last = nout - 1
        lax.fori_loop(0, last, body, 0)
        for b in range(_NBUF):
            gwait(b)
            ostart(b, last * _NBUF + b)
        for b in range(_NBUF):
            owait(b)

    return k(table_lin, idx_g)


def kernel(idx, embedding_weight):
    idx_g = (
        idx.reshape(32, 128, PAIRS, 2)
        .transpose(2, 0, 3, 1)
        .reshape(_NGRP, _G)
    )
    perm = jnp.arange(_G) // 2 + (jnp.arange(_G) % 2) * EMBED_DIM
    idx_g = idx_g[:, perm]
    idx_g = 2 * idx_g - jnp.where(idx_g >= _H, 2 * _H - 1, 0)

    table_t = jnp.transpose(embedding_weight)
    table_t2 = table_t[:, _H:]
    table_lin = _detile(table_t, table_t2).reshape(2 * _H, EMBED_DIM)
    nh = _NGRP // 2
    gat_a = _sc_gather(table_lin, idx_g[:nh])
    gat_b = _sc_gather(table_lin, idx_g[nh:])
    out_a = _tile_xpose_a(gat_a.reshape(_HP, 32, 2, EMBED_DIM, 128))
    out6 = _tile_xpose_b(gat_b.reshape(_HP, 32, 2, EMBED_DIM, 128), out_a)
    return (
        out6.transpose(3, 5, 0, 1, 2, 4)
        .reshape(BATCH, PAIRS, 2, EMBED_DIM)
    )
